# Initial kernel scaffold; baseline (speedup 1.0000x reference)
import jax
import jax.numpy as jnp
from jax.experimental import pallas as pl


def kernel(nodes, neigh_idx, features, W):
    raise NotImplementedError("write your pallas kernel here")


# speedup vs baseline: 3.1201x; 3.1201x over previous
import functools

import jax
import jax.numpy as jnp
from jax import lax
from jax.experimental import pallas as pl
from jax.experimental.pallas import tpu as pltpu
from jax.experimental.pallas import tpu_sc as plsc

_B = 100000
_D = 128
_K = 5
_NW = 32
_CN = 80
_NCHUNK = _B // _CN
_GSUB = 4
_GS = _CN * _K // _GSUB

_BK = 2048


def _gather_mean(features, idx3):
    mesh = plsc.VectorSubcoreMesh(core_axis_name="c", subcore_axis_name="s")

    @functools.partial(
        pl.kernel,
        out_type=jax.ShapeDtypeStruct((_B, _D), jnp.float32),
        mesh=mesh,
        scratch_types=[
            pltpu.VMEM((_GSUB, _GS), jnp.int32),
            pltpu.VMEM((_CN * _K, _D), jnp.float32),
            pltpu.VMEM((_CN, _D), jnp.float32),
            pltpu.SemaphoreType.DMA,
        ],
    )
    def k(feat_hbm, idx_hbm, out_hbm, idx_v, rows_v, out_v, sem):
        wid = lax.axis_index("s") * 2 + lax.axis_index("c")
        niter = (_NCHUNK - wid + _NW - 1) // _NW

        def chunk_body(i, _):
            g = wid + i * _NW
            pltpu.sync_copy(idx_hbm.at[g], idx_v)
            cps = [
                pltpu.async_copy(
                    feat_hbm.at[idx_v.at[s]],
                    rows_v.at[pl.ds(s * _GS, _GS)],
                    sem,
                )
                for s in range(_GSUB)
            ]
            for cp in cps:
                cp.wait()

            def node_body(n, _):
                r = n * _K
                for l in range(_D // 16):
                    sl = pl.ds(l * 16, 16)
                    acc = rows_v[r, sl]
                    for j in range(1, _K):
                        acc = acc + rows_v[r + j, sl]
                    out_v[n, sl] = acc * jnp.float32(1.0 / _K)
                return 0

            lax.fori_loop(0, _CN, node_body, 0)
            pltpu.sync_copy(out_v, out_hbm.at[pl.ds(g * _CN, _CN)])
            return 0

        lax.fori_loop(0, niter, chunk_body, 0)

    return k(features, idx3)


def _matmul_relu(W, mean_feats):
    def body(w_ref, x_ref, o_ref):
        y = lax.dot_general(
            w_ref[...], x_ref[...],
            (((1,), (1,)), ((), ())),
            preferred_element_type=jnp.float32,
        )
        o_ref[...] = jnp.maximum(y, 0.0)

    return pl.pallas_call(
        body,
        grid=((_B + _BK - 1) // _BK,),
        in_specs=[
            pl.BlockSpec((_D, _D), lambda i: (0, 0)),
            pl.BlockSpec((_BK, _D), lambda i: (i, 0)),
        ],
        out_specs=pl.BlockSpec((_D, _BK), lambda i: (0, i)),
        out_shape=jax.ShapeDtypeStruct((_D, _B), jnp.float32),
    )(W, mean_feats)


def kernel(nodes, neigh_idx, features, W):
    del nodes
    idx3 = neigh_idx.astype(jnp.int32).reshape(_NCHUNK, _GSUB, _GS)
    mean_feats = _gather_mean(features, idx3)
    return _matmul_relu(W, mean_feats)


# speedup vs baseline: 3.9545x; 1.2674x over previous
import functools

import jax
import jax.numpy as jnp
from jax import lax
from jax.experimental import pallas as pl
from jax.experimental.pallas import tpu as pltpu
from jax.experimental.pallas import tpu_sc as plsc

_B = 100000
_D = 128
_K = 5
_NW = 32
_CN = 80
_NCHUNK = _B // _CN
_GSUB = 4
_GS = _CN * _K // _GSUB

_BK = 2048


_MAXITER = (_NCHUNK + _NW - 1) // _NW


def _gather_mean(features, idx3):
    mesh = plsc.VectorSubcoreMesh(core_axis_name="c", subcore_axis_name="s")

    @functools.partial(
        pl.kernel,
        out_type=jax.ShapeDtypeStruct((_B, _D), jnp.float32),
        mesh=mesh,
        scratch_types=[
            pltpu.VMEM((_GSUB, _GS), jnp.int32),
            pltpu.VMEM((_GSUB, _GS), jnp.int32),
            pltpu.VMEM((_CN * _K, _D), jnp.float32),
            pltpu.VMEM((_CN * _K, _D), jnp.float32),
            pltpu.VMEM((_CN, _D), jnp.float32),
            pltpu.VMEM((_CN, _D), jnp.float32),
            pltpu.SemaphoreType.DMA,
            pltpu.SemaphoreType.DMA,
            pltpu.SemaphoreType.DMA,
            pltpu.SemaphoreType.DMA,
        ],
    )
    def k(feat_hbm, idx_hbm, out_hbm, idx_a, idx_b, rows_a, rows_b,
          out_a, out_b, sem_ga, sem_gb, sem_wa, sem_wb):
        wid = lax.axis_index("s") * 2 + lax.axis_index("c")
        idx_v = [idx_a, idx_b]
        rows_v = [rows_a, rows_b]
        out_v = [out_a, out_b]
        sem_g = [sem_ga, sem_gb]
        sem_w = [sem_wa, sem_wb]

        def fire_gathers(g, buf):
            pltpu.sync_copy(idx_hbm.at[g], idx_v[buf])
            for s in range(_GSUB):
                pltpu.async_copy(
                    feat_hbm.at[idx_v[buf].at[s]],
                    rows_v[buf].at[pl.ds(s * _GS, _GS)],
                    sem_g[buf],
                )

        def wait_gathers(buf):
            for s in range(_GSUB):
                pltpu.make_async_copy(
                    feat_hbm.at[idx_v[buf].at[s]],
                    rows_v[buf].at[pl.ds(s * _GS, _GS)],
                    sem_g[buf],
                ).wait()

        def compute(buf):
            rows, out = rows_v[buf], out_v[buf]

            def node_body(n, _):
                r = n * _K
                for l in range(_D // 16):
                    sl = pl.ds(l * 16, 16)
                    acc = rows[r, sl]
                    for j in range(1, _K):
                        acc = acc + rows[r + j, sl]
                    out[n, sl] = acc * jnp.float32(1.0 / _K)
                return 0

            lax.fori_loop(0, _CN, node_body, 0)

        def out_copy(g, buf):
            return pltpu.make_async_copy(
                out_v[buf], out_hbm.at[pl.ds(g * _CN, _CN)], sem_w[buf])

        fire_gathers(wid, 0)

        def outer(ii, _):
            for b in (0, 1):
                i_cur = ii * 2 + b
                g_cur = wid + i_cur * _NW
                g_next = g_cur + _NW

                @pl.when(g_next < _NCHUNK)
                def _prefetch():
                    fire_gathers(g_next, 1 - b)

                @pl.when(g_cur < _NCHUNK)
                def _work():
                    wait_gathers(b)

                    @pl.when(i_cur >= 2)
                    def _drain_prev():
                        out_copy(g_cur, b).wait()

                    compute(b)
                    out_copy(g_cur, b).start()
            return 0

        lax.fori_loop(0, (_MAXITER + 1) // 2, outer, 0)
        out_copy(wid, 0).wait()
        out_copy(wid, 1).wait()

    return k(features, idx3)


def _matmul_relu(W, mean_feats):
    def body(w_ref, x_ref, o_ref):
        y = lax.dot_general(
            w_ref[...], x_ref[...],
            (((1,), (1,)), ((), ())),
            preferred_element_type=jnp.float32,
        )
        o_ref[...] = jnp.maximum(y, 0.0)

    return pl.pallas_call(
        body,
        grid=((_B + _BK - 1) // _BK,),
        in_specs=[
            pl.BlockSpec((_D, _D), lambda i: (0, 0)),
            pl.BlockSpec((_BK, _D), lambda i: (i, 0)),
        ],
        out_specs=pl.BlockSpec((_D, _BK), lambda i: (0, i)),
        out_shape=jax.ShapeDtypeStruct((_D, _B), jnp.float32),
    )(W, mean_feats)


def kernel(nodes, neigh_idx, features, W):
    del nodes
    idx3 = neigh_idx.astype(jnp.int32).reshape(_NCHUNK, _GSUB, _GS)
    mean_feats = _gather_mean(features, idx3)
    return _matmul_relu(W, mean_feats)


# speedup vs baseline: 6.1178x; 1.5470x over previous
import functools

import jax
import jax.numpy as jnp
from jax import lax
from jax.experimental import pallas as pl
from jax.experimental.pallas import tpu as pltpu
from jax.experimental.pallas import tpu_sc as plsc

_B = 100000
_D = 128
_K = 5
_NW = 32
_CN = 80
_NCHUNK = _B // _CN
_GSUB = 4
_GS = _CN * _K // _GSUB

_BK = 2048


_MAXITER = (_NCHUNK + _NW - 1) // _NW


def _gather_mean(features, idx3):
    mesh = plsc.VectorSubcoreMesh(core_axis_name="c", subcore_axis_name="s")

    @functools.partial(
        pl.kernel,
        out_type=jax.ShapeDtypeStruct((_B, _D), jnp.float32),
        mesh=mesh,
        scratch_types=[
            pltpu.VMEM((_GSUB, _GS), jnp.int32),
            pltpu.VMEM((_GSUB, _GS), jnp.int32),
            pltpu.VMEM((_CN * _K, _D), jnp.float32),
            pltpu.VMEM((_CN * _K, _D), jnp.float32),
            pltpu.VMEM((_CN, _D), jnp.float32),
            pltpu.VMEM((_CN, _D), jnp.float32),
            pltpu.SemaphoreType.DMA,
            pltpu.SemaphoreType.DMA,
            pltpu.SemaphoreType.DMA,
            pltpu.SemaphoreType.DMA,
        ],
    )
    def k(feat_hbm, idx_hbm, out_hbm, idx_a, idx_b, rows_a, rows_b,
          out_a, out_b, sem_ga, sem_gb, sem_wa, sem_wb):
        wid = lax.axis_index("s") * 2 + lax.axis_index("c")
        idx_v = [idx_a, idx_b]
        rows_v = [rows_a, rows_b]
        out_v = [out_a, out_b]
        sem_g = [sem_ga, sem_gb]
        sem_w = [sem_wa, sem_wb]

        def fire_gathers(g, buf):
            pltpu.sync_copy(idx_hbm.at[g], idx_v[buf])
            for s in range(_GSUB):
                pltpu.async_copy(
                    feat_hbm.at[idx_v[buf].at[s]],
                    rows_v[buf].at[pl.ds(s * _GS, _GS)],
                    sem_g[buf],
                )

        def wait_gathers(buf):
            for s in range(_GSUB):
                pltpu.make_async_copy(
                    feat_hbm.at[idx_v[buf].at[s]],
                    rows_v[buf].at[pl.ds(s * _GS, _GS)],
                    sem_g[buf],
                ).wait()

        def compute(buf):
            rows, out = rows_v[buf], out_v[buf]

            @plsc.parallel_loop(0, _CN, unroll=2)
            def node_body(n):
                r = n * _K
                for l in range(_D // 16):
                    sl = pl.ds(l * 16, 16)
                    acc = rows[r, sl]
                    for j in range(1, _K):
                        acc = acc + rows[r + j, sl]
                    out[n, sl] = acc * jnp.float32(1.0 / _K)

        def out_copy(g, buf):
            return pltpu.make_async_copy(
                out_v[buf], out_hbm.at[pl.ds(g * _CN, _CN)], sem_w[buf])

        fire_gathers(wid, 0)

        def outer(ii, _):
            for b in (0, 1):
                i_cur = ii * 2 + b
                g_cur = wid + i_cur * _NW
                g_next = g_cur + _NW

                @pl.when(g_next < _NCHUNK)
                def _prefetch():
                    fire_gathers(g_next, 1 - b)

                @pl.when(g_cur < _NCHUNK)
                def _work():
                    wait_gathers(b)

                    @pl.when(i_cur >= 2)
                    def _drain_prev():
                        out_copy(g_cur, b).wait()

                    compute(b)
                    out_copy(g_cur, b).start()
            return 0

        lax.fori_loop(0, (_MAXITER + 1) // 2, outer, 0)
        out_copy(wid, 0).wait()
        out_copy(wid, 1).wait()

    return k(features, idx3)


def _matmul_relu(W, mean_feats):
    def body(w_ref, x_ref, o_ref):
        y = lax.dot_general(
            w_ref[...], x_ref[...],
            (((1,), (1,)), ((), ())),
            preferred_element_type=jnp.float32,
        )
        o_ref[...] = jnp.maximum(y, 0.0)

    return pl.pallas_call(
        body,
        grid=((_B + _BK - 1) // _BK,),
        in_specs=[
            pl.BlockSpec((_D, _D), lambda i: (0, 0)),
            pl.BlockSpec((_BK, _D), lambda i: (i, 0)),
        ],
        out_specs=pl.BlockSpec((_D, _BK), lambda i: (0, i)),
        out_shape=jax.ShapeDtypeStruct((_D, _B), jnp.float32),
    )(W, mean_feats)


def kernel(nodes, neigh_idx, features, W):
    del nodes
    idx3 = neigh_idx.astype(jnp.int32).reshape(_NCHUNK, _GSUB, _GS)
    mean_feats = _gather_mean(features, idx3)
    return _matmul_relu(W, mean_feats)
